# baseline (speedup 1.0000x reference)
import functools

import numpy as np
import jax
import jax.numpy as jnp
from jax.experimental import pallas as pl
from jax.experimental.pallas import tpu as pltpu
from jax.sharding import Mesh, PartitionSpec as P

try:
    from jax.experimental.shard_map import shard_map as _shard_map
except ImportError:
    from jax import shard_map as _shard_map

N = 4096
F = 512
H = 512
NOUT = 128
RB = 512


def _bdot(a, b):
    return jnp.dot(a.astype(jnp.bfloat16), b.astype(jnp.bfloat16),
                   preferred_element_type=jnp.float32)


def _enc_body(x_ref, w1_ref, b1_ref, w2_ref, b2_ref, gw_ref, s_ref):
    h = jnp.maximum(_bdot(x_ref[...], w1_ref[...]) + b1_ref[...], 0.0)
    h = jnp.maximum(_bdot(h, w2_ref[...]) + b2_ref[...], 0.0)
    s_ref[...] = _bdot(h, gw_ref[...]).astype(jnp.bfloat16)


def _agg_support_body(adj_ref, s_ref, gcb_ref, gw_ref, out_ref):
    a16 = adj_ref[...].astype(jnp.bfloat16)
    agg = jnp.dot(a16, s_ref[...], preferred_element_type=jnp.float32)
    h = jnp.maximum(agg + gcb_ref[...], 0.0)
    out_ref[...] = _bdot(h, gw_ref[...]).astype(jnp.bfloat16)


def _agg_final_body(adj_ref, s_ref, gcb_ref, wz_ref, emb_ref, zp_ref, acc_ref):
    i = pl.program_id(0)
    a16 = adj_ref[...].astype(jnp.bfloat16)
    agg = jnp.dot(a16, s_ref[...], preferred_element_type=jnp.float32)
    h = jnp.maximum(agg + gcb_ref[...], 0.0)
    emb_ref[...] = h
    colsum = jnp.sum(h, axis=0, keepdims=True)

    @pl.when(i == 0)
    def _init():
        acc_ref[...] = colsum

    @pl.when(i > 0)
    def _accum():
        acc_ref[...] = acc_ref[...] + colsum

    @pl.when(i == pl.num_programs(0) - 1)
    def _decode():
        mean = acc_ref[...] * (1.0 / N)
        zp_ref[...] = _bdot(mean, wz_ref[...])


def _full(*shape):
    return pl.BlockSpec(shape, lambda i: (0,) * len(shape))


def _pipeline(x2d, adj_rows, W1, b1r, W2, b2r, gcW, gcbr, We2p, axis_name):
    n_loc = x2d.shape[0]
    nb = n_loc // RB

    s1 = pl.pallas_call(
        _enc_body,
        grid=(nb,),
        in_specs=[
            pl.BlockSpec((RB, F), lambda i: (i, 0)),
            _full(F, H), _full(1, H), _full(H, H), _full(1, H), _full(H, H),
        ],
        out_specs=pl.BlockSpec((RB, H), lambda i: (i, 0)),
        out_shape=jax.ShapeDtypeStruct((n_loc, H), jnp.bfloat16),
    )(x2d, W1, b1r, W2, b2r, gcW)

    if axis_name is not None:
        s1 = jax.lax.all_gather(s1, axis_name, axis=0, tiled=True)

    s2 = pl.pallas_call(
        _agg_support_body,
        grid=(nb,),
        in_specs=[
            pl.BlockSpec((RB, N), lambda i: (i, 0)),
            _full(N, H), _full(1, H), _full(H, H),
        ],
        out_specs=pl.BlockSpec((RB, H), lambda i: (i, 0)),
        out_shape=jax.ShapeDtypeStruct((n_loc, H), jnp.bfloat16),
    )(adj_rows, s1, gcbr, gcW)

    if axis_name is not None:
        s2 = jax.lax.all_gather(s2, axis_name, axis=0, tiled=True)

    emb, zp = pl.pallas_call(
        _agg_final_body,
        grid=(nb,),
        in_specs=[
            pl.BlockSpec((RB, N), lambda i: (i, 0)),
            _full(N, H), _full(1, H), _full(H, NOUT),
        ],
        out_specs=[
            pl.BlockSpec((RB, H), lambda i: (i, 0)),
            pl.BlockSpec((1, NOUT), lambda i: (0, 0)),
        ],
        out_shape=[
            jax.ShapeDtypeStruct((n_loc, H), jnp.float32),
            jax.ShapeDtypeStruct((1, NOUT), jnp.float32),
        ],
        scratch_shapes=[pltpu.VMEM((1, H), jnp.float32)],
    )(adj_rows, s2, gcbr, We2p)

    if axis_name is not None:
        zp = jax.lax.psum(zp, axis_name)
    return emb, zp


def kernel(inputs, adj, W1, b1, W2, b2, gcW, gcb, We2p, be2p):
    x2d = inputs.reshape(N, F)
    b1r = b1.reshape(1, H)
    b2r = b2.reshape(1, H)
    gcbr = gcb.reshape(1, H)

    devs = jax.devices()
    nd = 1
    while nd * 2 <= min(len(devs), 8) and (N // (nd * 2)) % RB == 0:
        nd *= 2

    if nd == 1:
        emb, zp = _pipeline(x2d, adj, W1, b1r, W2, b2r, gcW, gcbr, We2p, None)
    else:
        mesh = Mesh(np.array(devs[:nd]), ("x",))
        fn = functools.partial(_pipeline, axis_name="x")
        emb, zp = _shard_map(
            fn,
            mesh=mesh,
            in_specs=(P("x", None), P("x", None), P(None, None), P(None, None),
                      P(None, None), P(None, None), P(None, None), P(None, None),
                      P(None, None)),
            out_specs=(P("x", None), P(None, None)),
            check_rep=False,
        )(x2d, adj, W1, b1r, W2, b2r, gcW, gcbr, We2p)

    z = zp + be2p.reshape(1, NOUT)
    return (emb.reshape(1, N, H), z)


# speedup vs baseline: 9.1685x; 9.1685x over previous
import jax
import jax.numpy as jnp
from jax.experimental import pallas as pl
from jax.experimental.pallas import tpu as pltpu

N = 4096
F = 512
H = 512
NOUT = 128
RB = 256
NB = N // RB


def _mega_body(x_ref, adj_ref, w1_ref, b1_ref, w2_ref, b2_ref, gw_ref,
               gcb_ref, wz_ref, bz_ref, emb_ref, z_ref,
               adjc_ref, s1_ref, s2_ref, acc_ref):
    i = pl.program_id(0)

    @pl.when(i < NB)
    def _encode():
        x16 = x_ref[...].astype(jnp.bfloat16)
        h = jnp.dot(x16, w1_ref[...].astype(jnp.bfloat16),
                    preferred_element_type=jnp.float32)
        h = jnp.maximum(h + b1_ref[...], 0.0)
        h = jnp.dot(h.astype(jnp.bfloat16), w2_ref[...].astype(jnp.bfloat16),
                    preferred_element_type=jnp.float32)
        h = jnp.maximum(h + b2_ref[...], 0.0)
        s = jnp.dot(h.astype(jnp.bfloat16), gw_ref[...].astype(jnp.bfloat16),
                    preferred_element_type=jnp.float32)
        s1_ref[pl.ds(i * RB, RB), :] = s.astype(jnp.bfloat16)

    @pl.when(jnp.logical_and(i >= NB, i < 2 * NB))
    def _iter1():
        j = i - NB
        a16 = adj_ref[...].astype(jnp.bfloat16)
        adjc_ref[pl.ds(j * RB, RB), :] = a16
        agg = jnp.dot(a16, s1_ref[...], preferred_element_type=jnp.float32)
        h = jnp.maximum(agg + gcb_ref[...], 0.0)
        s = jnp.dot(h.astype(jnp.bfloat16), gw_ref[...].astype(jnp.bfloat16),
                    preferred_element_type=jnp.float32)
        s2_ref[pl.ds(j * RB, RB), :] = s.astype(jnp.bfloat16)

    @pl.when(i >= 2 * NB)
    def _iter2():
        k = i - 2 * NB
        a16 = adjc_ref[pl.ds(k * RB, RB), :]
        agg = jnp.dot(a16, s2_ref[...], preferred_element_type=jnp.float32)
        h = jnp.maximum(agg + gcb_ref[...], 0.0)
        emb_ref[...] = h
        colsum = jnp.sum(h, axis=0, keepdims=True)

        @pl.when(k == 0)
        def _init():
            acc_ref[...] = colsum

        @pl.when(k > 0)
        def _accum():
            acc_ref[...] = acc_ref[...] + colsum

        @pl.when(k == NB - 1)
        def _decode():
            mean = (acc_ref[...] * (1.0 / N)).astype(jnp.bfloat16)
            z_ref[...] = (
                jnp.dot(mean, wz_ref[...].astype(jnp.bfloat16),
                        preferred_element_type=jnp.float32)
                + bz_ref[...]
            )


def _full(*shape):
    return pl.BlockSpec(shape, lambda i: (0,) * len(shape))


def kernel(inputs, adj, W1, b1, W2, b2, gcW, gcb, We2p, be2p):
    x2d = inputs.reshape(N, F)
    b1r = b1.reshape(1, H)
    b2r = b2.reshape(1, H)
    gcbr = gcb.reshape(1, H)
    bzr = be2p.reshape(1, NOUT)

    emb, z = pl.pallas_call(
        _mega_body,
        grid=(3 * NB,),
        in_specs=[
            pl.BlockSpec((RB, F), lambda i: (jnp.clip(i, 0, NB - 1), 0)),
            pl.BlockSpec((RB, N), lambda i: (jnp.clip(i - NB, 0, NB - 1), 0)),
            _full(F, H), _full(1, H), _full(H, H), _full(1, H), _full(H, H),
            _full(1, H), _full(H, NOUT), _full(1, NOUT),
        ],
        out_specs=[
            pl.BlockSpec((RB, H), lambda i: (jnp.clip(i - 2 * NB, 0, NB - 1), 0)),
            pl.BlockSpec((1, NOUT), lambda i: (0, 0)),
        ],
        out_shape=[
            jax.ShapeDtypeStruct((N, H), jnp.float32),
            jax.ShapeDtypeStruct((1, NOUT), jnp.float32),
        ],
        scratch_shapes=[
            pltpu.VMEM((N, N), jnp.bfloat16),
            pltpu.VMEM((N, H), jnp.bfloat16),
            pltpu.VMEM((N, H), jnp.bfloat16),
            pltpu.VMEM((1, H), jnp.float32),
        ],
    )(x2d, adj, W1, b1r, W2, b2r, gcW, gcbr, We2p, bzr)

    return (emb.reshape(1, N, H), z)
